# Initial kernel scaffold; baseline (speedup 1.0000x reference)
import jax
import jax.numpy as jnp
from jax.experimental import pallas as pl


def kernel(x, edge_index, batch, W1, b1, W2, b2):
    raise NotImplementedError("write your pallas kernel here")


# speedup vs baseline: 8.7732x; 8.7732x over previous
import functools

import jax
import jax.numpy as jnp
from jax import lax
from jax.experimental import pallas as pl
from jax.experimental.pallas import tpu as pltpu
from jax.experimental.pallas import tpu_sc as plsc

_BIG = 1e30
_K = 16
_BR = 256


def _leaky(v):
    return jnp.where(v >= 0, v, 0.01 * v)


def _knn_body(x_ref, pos_ref, posT_ref, brow_ref, bcol_ref, wc_ref, bc_ref,
              idx_ref, a_ref, b_ref):
    i = pl.program_id(0)
    n = posT_ref.shape[1]
    l2 = a_ref.shape[1]

    ab = jnp.dot(x_ref[...], wc_ref[...], preferred_element_type=jnp.float32)
    ab = ab + bc_ref[0:1, :]
    a_ref[...] = ab[:, :l2]
    b_ref[...] = ab[:, l2:]

    pos = pos_ref[...]
    pT = posT_ref[...]
    sqr = jnp.sum(pos * pos, axis=1, keepdims=True)
    sqc = jnp.sum(pT * pT, axis=0, keepdims=True)
    d2 = (sqr + sqc) - 2.0 * jnp.dot(pos, pT, preferred_element_type=jnp.float32)

    cols = lax.broadcasted_iota(jnp.int32, (_BR, n), 1)
    rows = i * _BR + lax.broadcasted_iota(jnp.int32, (_BR, n), 0)
    invalid = (brow_ref[...] != bcol_ref[0:1, :]) | (cols == rows)
    d2 = jnp.where(invalid, _BIG, d2)

    picked = []
    for _ in range(_K):
        v = jnp.min(d2, axis=1, keepdims=True)
        cand = jnp.where(d2 == v, cols, jnp.int32(n))
        c = jnp.min(cand, axis=1, keepdims=True)
        picked.append(c)
        d2 = jnp.where(cols == c, float("inf"), d2)
    idx_ref[...] = jnp.concatenate(picked, axis=1)


def _edge_body(a_ref, g_ref, w2_ref, b2_ref, o_ref):
    a = a_ref[...]
    w2 = w2_ref[...]
    b2 = b2_ref[0:1, :]
    mx = None
    ms = None
    for k in range(_K):
        h = _leaky(g_ref[k] + a)
        m = _leaky(jnp.dot(h, w2, preferred_element_type=jnp.float32) + b2)
        mx = m if k == 0 else jnp.maximum(mx, m)
        ms = m if k == 0 else ms + m
    o_ref[...] = _leaky(jnp.concatenate([mx, ms * (1.0 / _K), ms], axis=1))


def _sc_gather(table, idxp, n_rows, d):
    info = plsc.get_sparse_core_info()
    nw = info.num_cores * info.num_subcores
    epw = n_rows // nw
    ch = 128
    mesh = plsc.VectorSubcoreMesh(core_axis_name="c", subcore_axis_name="s")

    @functools.partial(
        pl.kernel,
        mesh=mesh,
        out_type=jax.ShapeDtypeStruct((n_rows, d), jnp.float32),
        scratch_types=[
            pltpu.VMEM((ch,), jnp.int32),
            pltpu.VMEM((ch, d), jnp.float32),
            pltpu.SemaphoreType.DMA,
        ],
    )
    def gather_k(table_hbm, idx_hbm, out_hbm, idx_v, rows_v, sem):
        wid = lax.axis_index("s") * info.num_cores + lax.axis_index("c")
        base = wid * epw

        def body(j, carry):
            off = base + j * ch
            pltpu.sync_copy(idx_hbm.at[pl.ds(off, ch)], idx_v)
            pltpu.async_copy(table_hbm.at[idx_v], rows_v, sem).wait()
            pltpu.sync_copy(rows_v, out_hbm.at[pl.ds(off, ch)])
            return carry

        lax.fori_loop(0, epw // ch, body, 0)

    return gather_k(table, idxp)


def kernel(x, edge_index, batch, W1, b1, W2, b2):
    n, l1 = x.shape
    l2 = W1.shape[1]
    l3 = W2.shape[1]
    nb = n // _BR

    pos_pad = jnp.pad(x[:, :3], ((0, 0), (0, 5)))
    posT = pos_pad.T
    batch_row = batch.reshape(n, 1)
    batch_col = jnp.broadcast_to(batch.reshape(1, n), (8, n))
    wc = jnp.concatenate([W1[:l1] - W1[l1:], W1[l1:]], axis=1)
    bc = jnp.broadcast_to(
        jnp.concatenate([b1, jnp.zeros_like(b1)]).reshape(1, 2 * l2), (8, 2 * l2))
    b2p = jnp.broadcast_to(b2.reshape(1, l3), (8, l3))

    idx, a_arr, b_arr = pl.pallas_call(
        _knn_body,
        grid=(nb,),
        in_specs=[
            pl.BlockSpec((_BR, l1), lambda i: (i, 0)),
            pl.BlockSpec((_BR, 8), lambda i: (i, 0)),
            pl.BlockSpec((8, n), lambda i: (0, 0)),
            pl.BlockSpec((_BR, 1), lambda i: (i, 0)),
            pl.BlockSpec((8, n), lambda i: (0, 0)),
            pl.BlockSpec((l1, 2 * l2), lambda i: (0, 0)),
            pl.BlockSpec((8, 2 * l2), lambda i: (0, 0)),
        ],
        out_specs=[
            pl.BlockSpec((_BR, _K), lambda i: (i, 0)),
            pl.BlockSpec((_BR, l2), lambda i: (i, 0)),
            pl.BlockSpec((_BR, l2), lambda i: (i, 0)),
        ],
        out_shape=[
            jax.ShapeDtypeStruct((n, _K), jnp.int32),
            jax.ShapeDtypeStruct((n, l2), jnp.float32),
            jax.ShapeDtypeStruct((n, l2), jnp.float32),
        ],
    )(x, pos_pad, posT, batch_row, batch_col, wc, bc)

    idx_perm = idx.T.reshape(-1)
    g = _sc_gather(b_arr, idx_perm, n * _K, l2).reshape(_K, n, l2)

    out = pl.pallas_call(
        _edge_body,
        grid=(nb,),
        in_specs=[
            pl.BlockSpec((_BR, l2), lambda i: (i, 0)),
            pl.BlockSpec((_K, _BR, l2), lambda i: (0, i, 0)),
            pl.BlockSpec((l2, l3), lambda i: (0, 0)),
            pl.BlockSpec((8, l3), lambda i: (0, 0)),
        ],
        out_specs=pl.BlockSpec((_BR, 3 * l3), lambda i: (i, 0)),
        out_shape=jax.ShapeDtypeStruct((n, 3 * l3), jnp.float32),
    )(a_arr, g, W2, b2p)

    rows = jnp.repeat(jnp.arange(n, dtype=idx.dtype), _K)
    new_edge_index = jnp.stack([idx.reshape(-1), rows], axis=0)
    return (out, new_edge_index)
